# baseline (speedup 1.0000x reference)
import functools

import jax
import jax.numpy as jnp
from jax import lax
from jax.experimental import pallas as pl
from jax.experimental.pallas import tpu as pltpu
from jax.experimental.pallas import tpu_sc as plsc

N = 2097152
NW = 32
PER_W = N // NW
C = 16384
NCHUNK = PER_W // C

ONE_M_EPS = 1.0 - float(jnp.finfo(jnp.float32).eps)

_mesh = plsc.VectorSubcoreMesh(core_axis_name="c", subcore_axis_name="s")


def _pick48(a, b, c, src):
    ia = jnp.minimum(src, 15)
    ib = jnp.clip(src - 16, 0, 15)
    ic = jnp.clip(src - 32, 0, 15)
    ga = a.at[ia].get(mode="promise_in_bounds")
    gb = b.at[ib].get(mode="promise_in_bounds")
    gc = c.at[ic].get(mode="promise_in_bounds")
    return jnp.where(src < 16, ga, jnp.where(src < 32, gb, gc))


@functools.partial(
    pl.kernel,
    mesh=_mesh,
    out_type=jax.ShapeDtypeStruct((N,), jnp.float32),
    scratch_types=[
        pltpu.VMEM((3 * C,), jnp.float32),
        pltpu.VMEM((C,), jnp.int32),
        pltpu.VMEM((C,), jnp.float32),
        pltpu.SemaphoreType.DMA,
    ],
)
def _grid_gather(x_hbm, grid_hbm, out_hbm, xall, idxv, outv, sem):
    wid = lax.axis_index("s") * 2 + lax.axis_index("c")
    base = wid * PER_W
    lane3 = lax.iota(jnp.int32, 16) * 3

    def chunk_body(ci, carry):
        cbase = base + ci * C
        pltpu.sync_copy(x_hbm.at[pl.ds(3 * cbase, 3 * C)], xall)

        def vec_body(i, carry2):
            s = i * 48
            a = xall[pl.ds(s, 16)]
            b = xall[pl.ds(s + 16, 16)]
            c = xall[pl.ds(s + 32, 16)]
            x0 = _pick48(a, b, c, lane3)
            x1 = _pick48(a, b, c, lane3 + 1)
            x2 = _pick48(a, b, c, lane3 + 2)

            def to_cell(v):
                t = (v + 1.0) * 0.5
                t = jnp.minimum(jnp.maximum(t, 0.0), ONE_M_EPS)
                return (t * 256.0).astype(jnp.int32)

            lin = (to_cell(x0) << 16) | (to_cell(x1) << 8) | to_cell(x2)
            idxv[pl.ds(i * 16, 16)] = lin
            return carry2

        lax.fori_loop(0, C // 16, vec_body, 0, unroll=4)

        pltpu.async_copy(grid_hbm.at[idxv], outv, sem).wait()
        pltpu.sync_copy(outv, out_hbm.at[pl.ds(cbase, C)])
        return carry

    lax.fori_loop(0, NCHUNK, chunk_body, 0)


def kernel(x, grid):
    return _grid_gather(x.reshape(-1), grid.reshape(-1))


# speedup vs baseline: 1.0607x; 1.0607x over previous
import functools

import jax
import jax.numpy as jnp
from jax import lax
from jax.experimental import pallas as pl
from jax.experimental.pallas import tpu as pltpu
from jax.experimental.pallas import tpu_sc as plsc

N = 2097152
ROWS = (3 * N) // 384
RB = 1024

NW = 32
PER_W = N // NW
C = 32768
NCHUNK = PER_W // C

ONE_M_EPS = 1.0 - float(jnp.finfo(jnp.float32).eps)

_mesh = plsc.VectorSubcoreMesh(core_axis_name="c", subcore_axis_name="s")


def _idx_body(x_ref, o_ref):
    xb = x_ref[...]
    t = (xb + 1.0) * 0.5
    t = jnp.minimum(jnp.maximum(t, 0.0), ONE_M_EPS) * 256.0
    cf = jnp.floor(t)
    j = lax.broadcasted_iota(jnp.int32, (384, 128), 0)
    q = lax.broadcasted_iota(jnp.int32, (384, 128), 1)
    d = j - 3 * q
    w = jnp.where(d == 0, 65536.0, jnp.where(d == 1, 256.0, 1.0))
    W = jnp.where((d >= 0) & (d < 3), w, 0.0)
    lin_f = jnp.dot(cf, W, preferred_element_type=jnp.float32)
    o_ref[...] = lin_f.astype(jnp.int32)


def _lin_idx(x_flat):
    x2d = x_flat.reshape(ROWS, 384)
    return pl.pallas_call(
        _idx_body,
        grid=(ROWS // RB,),
        in_specs=[pl.BlockSpec((RB, 384), lambda i: (i, 0))],
        out_specs=pl.BlockSpec((RB, 128), lambda i: (i, 0)),
        out_shape=jax.ShapeDtypeStruct((ROWS, 128), jnp.int32),
    )(x2d)


@functools.partial(
    pl.kernel,
    mesh=_mesh,
    out_type=jax.ShapeDtypeStruct((N,), jnp.float32),
    scratch_types=[
        pltpu.VMEM((C,), jnp.int32),
        pltpu.VMEM((C,), jnp.float32),
        pltpu.SemaphoreType.DMA,
    ],
)
def _grid_gather(idx_hbm, grid_hbm, out_hbm, idxv, outv, sem):
    wid = lax.axis_index("s") * 2 + lax.axis_index("c")
    base = wid * PER_W

    def chunk_body(ci, carry):
        cbase = base + ci * C
        pltpu.sync_copy(idx_hbm.at[pl.ds(cbase, C)], idxv)
        pltpu.async_copy(grid_hbm.at[idxv], outv, sem).wait()
        pltpu.sync_copy(outv, out_hbm.at[pl.ds(cbase, C)])
        return carry

    lax.fori_loop(0, NCHUNK, chunk_body, 0)


def kernel(x, grid):
    lin = _lin_idx(x.reshape(-1)).reshape(-1)
    return _grid_gather(lin, grid.reshape(-1))


# speedup vs baseline: 14.8542x; 14.0044x over previous
import functools

import jax
import jax.numpy as jnp
from jax import lax
from jax.experimental import pallas as pl
from jax.experimental.pallas import tpu as pltpu
from jax.experimental.pallas import tpu_sc as plsc

N = 2097152
ROWS = N // 128
RB = 2048

NW = 32
PER_W = N // NW
C = 32768
NCHUNK = PER_W // C

ONE_M_EPS = 1.0 - float(jnp.finfo(jnp.float32).eps)

_mesh = plsc.VectorSubcoreMesh(core_axis_name="c", subcore_axis_name="s")


def _idx_body(x0_ref, x1_ref, x2_ref, o_ref):
    def cell(v):
        t = (v + 1.0) * 0.5
        t = jnp.minimum(jnp.maximum(t, 0.0), ONE_M_EPS) * 256.0
        return t.astype(jnp.int32)

    lin = (
        (cell(x0_ref[...]) << 16)
        | (cell(x1_ref[...]) << 8)
        | cell(x2_ref[...])
    )
    o_ref[...] = lin


def _lin_idx(x0, x1, x2):
    spec = pl.BlockSpec((RB, 128), lambda i: (i, 0))
    return pl.pallas_call(
        _idx_body,
        grid=(ROWS // RB,),
        in_specs=[spec, spec, spec],
        out_specs=spec,
        out_shape=jax.ShapeDtypeStruct((ROWS, 128), jnp.int32),
    )(x0.reshape(ROWS, 128), x1.reshape(ROWS, 128), x2.reshape(ROWS, 128))


@functools.partial(
    pl.kernel,
    mesh=_mesh,
    out_type=jax.ShapeDtypeStruct((N,), jnp.float32),
    scratch_types=[
        pltpu.VMEM((C,), jnp.int32),
        pltpu.VMEM((C,), jnp.float32),
        pltpu.SemaphoreType.DMA,
    ],
)
def _grid_gather(idx_hbm, grid_hbm, out_hbm, idxv, outv, sem):
    wid = lax.axis_index("s") * 2 + lax.axis_index("c")
    base = wid * PER_W

    def chunk_body(ci, carry):
        cbase = base + ci * C
        pltpu.sync_copy(idx_hbm.at[pl.ds(cbase, C)], idxv)
        pltpu.async_copy(grid_hbm.at[idxv], outv, sem).wait()
        pltpu.sync_copy(outv, out_hbm.at[pl.ds(cbase, C)])
        return carry

    lax.fori_loop(0, NCHUNK, chunk_body, 0)


def kernel(x, grid):
    lin = _lin_idx(x[:, 0], x[:, 1], x[:, 2]).reshape(-1)
    return _grid_gather(lin, grid.reshape(-1))


# speedup vs baseline: 16.7394x; 1.1269x over previous
import functools

import jax
import jax.numpy as jnp
from jax import lax
from jax.experimental import pallas as pl
from jax.experimental.pallas import tpu as pltpu
from jax.experimental.pallas import tpu_sc as plsc

N = 2097152
BLKS = N // 128
RB = 2048

NW = 32
PER_W = N // NW
C = 32768
NCHUNK = PER_W // C

ONE_M_EPS = 1.0 - float(jnp.finfo(jnp.float32).eps)

_mesh = plsc.VectorSubcoreMesh(core_axis_name="c", subcore_axis_name="s")


def _idx_body(x_ref, o_ref):
    def cell(v):
        t = (v + 1.0) * 0.5
        t = jnp.minimum(jnp.maximum(t, 0.0), ONE_M_EPS) * 256.0
        return t.astype(jnp.int32)

    i = cell(x_ref[:, 0, :])
    j = cell(x_ref[:, 1, :])
    k = cell(x_ref[:, 2, :])
    phys = (
        (i << 16)
        | ((j >> 3) << 11)
        | ((k >> 7) << 10)
        | ((j & 7) << 7)
        | (k & 127)
    )
    o_ref[...] = phys


def _phys_idx(x3):
    return pl.pallas_call(
        _idx_body,
        grid=(BLKS // RB,),
        in_specs=[pl.BlockSpec((RB, 3, 128), lambda i: (i, 0, 0))],
        out_specs=pl.BlockSpec((RB, 128), lambda i: (i, 0)),
        out_shape=jax.ShapeDtypeStruct((BLKS, 128), jnp.int32),
    )(x3)


@functools.partial(
    pl.kernel,
    mesh=_mesh,
    out_type=jax.ShapeDtypeStruct((N,), jnp.float32),
    scratch_types=[
        pltpu.VMEM((C,), jnp.int32),
        pltpu.VMEM((C,), jnp.float32),
        pltpu.SemaphoreType.DMA,
    ],
)
def _grid_gather(idx_hbm, grid_hbm, out_hbm, idxv, outv, sem):
    wid = lax.axis_index("s") * 2 + lax.axis_index("c")
    base = wid * PER_W

    def chunk_body(ci, carry):
        cbase = base + ci * C
        pltpu.sync_copy(idx_hbm.at[pl.ds(cbase, C)], idxv)
        pltpu.async_copy(grid_hbm.at[idxv], outv, sem).wait()
        pltpu.sync_copy(outv, out_hbm.at[pl.ds(cbase, C)])
        return carry

    lax.fori_loop(0, NCHUNK, chunk_body, 0)


def kernel(x, grid):
    x3 = x.reshape(BLKS, 128, 3).transpose(0, 2, 1)
    grid_lin = (
        grid.reshape(256, 32, 8, 2, 128)
        .transpose(0, 1, 3, 2, 4)
        .reshape(-1)
    )
    lin = _phys_idx(x3).reshape(-1)
    return _grid_gather(lin, grid_lin)


# speedup vs baseline: 18.0323x; 1.0772x over previous
import functools

import jax
import jax.numpy as jnp
from jax import lax
from jax.experimental import pallas as pl
from jax.experimental.pallas import tpu as pltpu
from jax.experimental.pallas import tpu_sc as plsc

N = 2097152
NSLICE = 4
SL = N // NSLICE
SLB = SL // 128

NW = 32
C = SL // NW

MAX_CELL = 256.0 - 2.0 ** -15

_mesh = plsc.VectorSubcoreMesh(core_axis_name="c", subcore_axis_name="s")


def _idx_body(x_ref, o_ref):
    def cell(v):
        t = jnp.minimum(v * 128.0 + 128.0, MAX_CELL)
        return t.astype(jnp.int32)

    i = cell(x_ref[:, 0, :])
    j = cell(x_ref[:, 1, :])
    k = cell(x_ref[:, 2, :])
    jpart = (j << 7) + ((j >> 3) << 10)
    kpart = (k & 127) | ((k >> 7) << 10)
    o_ref[...] = (i << 16) | jpart | kpart


RB = SLB // 4


def _phys_idx(x3):
    return pl.pallas_call(
        _idx_body,
        grid=(SLB // RB,),
        in_specs=[pl.BlockSpec((RB, 3, 128), lambda i: (i, 0, 0))],
        out_specs=pl.BlockSpec((RB, 128), lambda i: (i, 0)),
        out_shape=jax.ShapeDtypeStruct((SLB, 128), jnp.int32),
    )(x3)


@functools.partial(
    pl.kernel,
    mesh=_mesh,
    out_type=jax.ShapeDtypeStruct((SL,), jnp.float32),
    scratch_types=[
        pltpu.VMEM((C,), jnp.int32),
        pltpu.VMEM((C,), jnp.float32),
        pltpu.SemaphoreType.DMA,
    ],
)
def _grid_gather(idx_hbm, grid_hbm, out_hbm, idxv, outv, sem):
    wid = lax.axis_index("s") * 2 + lax.axis_index("c")
    base = wid * C
    pltpu.sync_copy(idx_hbm.at[pl.ds(base, C)], idxv)
    pltpu.async_copy(grid_hbm.at[idxv], outv, sem).wait()
    pltpu.sync_copy(outv, out_hbm.at[pl.ds(base, C)])


def kernel(x, grid):
    x3 = x.reshape(N // 128, 128, 3).transpose(0, 2, 1)
    grid_lin = (
        grid.reshape(256, 32, 8, 2, 128)
        .transpose(0, 1, 3, 2, 4)
        .reshape(-1)
    )
    outs = []
    for s in range(NSLICE):
        lin = _phys_idx(x3[s * SLB:(s + 1) * SLB]).reshape(-1)
        outs.append(_grid_gather(lin, grid_lin))
    return jnp.concatenate(outs)


# speedup vs baseline: 18.7121x; 1.0377x over previous
import functools

import jax
import jax.numpy as jnp
from jax import lax
from jax.experimental import pallas as pl
from jax.experimental.pallas import tpu as pltpu
from jax.experimental.pallas import tpu_sc as plsc

N = 2097152
NSLICE = 4
SL = N // NSLICE
SLB = SL // 128

NW = 32
C = SL // NW

MAX_CELL = 256.0 - 2.0 ** -15

_mesh = plsc.VectorSubcoreMesh(core_axis_name="c", subcore_axis_name="s")


def _idx_body(x_ref, o_ref):
    def cell(v):
        t = jnp.minimum(v * 128.0 + 128.0, MAX_CELL)
        return t.astype(jnp.int32)

    i = cell(x_ref[:, 0, :])
    j = cell(x_ref[:, 1, :])
    k = cell(x_ref[:, 2, :])
    jpart = (j << 7) + ((j >> 3) << 10)
    kpart = (k & 127) | ((k >> 7) << 10)
    o_ref[...] = (i << 16) | jpart | kpart


RB = SLB // 4


def _phys_idx(x3):
    return pl.pallas_call(
        _idx_body,
        grid=(SLB // RB,),
        in_specs=[pl.BlockSpec((RB, 3, 128), lambda i: (i, 0, 0))],
        out_specs=pl.BlockSpec((RB, 128), lambda i: (i, 0)),
        out_shape=jax.ShapeDtypeStruct((SLB, 128), jnp.int32),
    )(x3)


def _make_gather(slice_base):
    @functools.partial(
        pl.kernel,
        mesh=_mesh,
        out_type=(),
        scratch_types=[
            pltpu.VMEM((C,), jnp.int32),
            pltpu.VMEM((C,), jnp.float32),
            pltpu.SemaphoreType.DMA,
        ],
    )
    def _gather(idx_hbm, grid_hbm, out_hbm, idxv, outv, sem):
        wid = lax.axis_index("s") * 2 + lax.axis_index("c")
        base = wid * C
        pltpu.sync_copy(idx_hbm.at[pl.ds(base, C)], idxv)
        pltpu.async_copy(grid_hbm.at[idxv], outv, sem).wait()
        pltpu.sync_copy(outv, out_hbm.at[pl.ds(slice_base + base, C)])

    return _gather


_gathers = [_make_gather(s * SL) for s in range(NSLICE)]


def kernel(x, grid):
    x3 = x.reshape(N // 128, 128, 3).transpose(0, 2, 1)
    grid_lin = (
        grid.reshape(256, 32, 8, 2, 128)
        .transpose(0, 1, 3, 2, 4)
        .reshape(-1)
    )
    out_ref = jax.new_ref(jnp.zeros((N,), jnp.float32))
    for s in range(NSLICE):
        lin = _phys_idx(x3[s * SLB:(s + 1) * SLB]).reshape(-1)
        _gathers[s](lin, grid_lin, out_ref)
    return out_ref[...]
